# baseline (speedup 1.0000x reference)
import jax
import jax.numpy as jnp
from jax.experimental import pallas as pl

O_N = 10000
NUM_OBJS_P1 = 180
NIMG = 8
EMB = 128
GDIM = 128
HID = 512
NOISE_DIM = 64

KPAD = 184
CROWS = KPAD + NIMG

_HI = jax.lax.Precision.HIGHEST


def _mono_kernel(objs_ref, oti_ref, noise_ref, obj_emb_ref, gconv_W_ref,
                 gconv_b_ref, att_W_ref, W1a_ref, W1b_ref, W1c_ref, b1_ref,
                 W2_ref, b2_ref, out_ref):
    objs_l = objs_ref[...]
    oti_l = oti_ref[...]
    iota = jax.lax.broadcasted_iota(jnp.int32, (CROWS, O_N), 0)
    cmb = ((iota == objs_l) | (iota == oti_l + KPAD)).astype(jnp.bfloat16)
    histT = jax.lax.dot_general(
        cmb[0:KPAD], cmb[KPAD:CROWS], (((1,), (1,)), ((), ())),
        preferred_element_type=jnp.float32)
    table_g = jnp.dot(obj_emb_ref[...], gconv_W_ref[...],
                      preferred_element_type=jnp.float32) + gconv_b_ref[...]
    table_a = jnp.dot(table_g, att_W_ref[...],
                      preferred_element_type=jnp.float32)
    counts = jax.lax.dot_general(
        histT, jnp.ones((KPAD, 1), jnp.float32),
        (((0,), (0,)), ((), ())), preferred_element_type=jnp.float32)
    counts = jnp.where(counts > 0.0, counts, 1.0)
    gc = jax.lax.dot_general(
        histT, table_a, (((0,), (0,)), ((), ())), precision=_HI,
        preferred_element_type=jnp.float32) / counts
    tg = jnp.tanh(gc)
    sig = jax.nn.sigmoid(jax.lax.dot_general(
        table_g, tg, (((1,), (1,)), ((), ())), precision=_HI,
        preferred_element_type=jnp.float32))
    w = histT * sig
    rep = jax.lax.dot_general(
        w, table_g, (((0,), (0,)), ((), ())), precision=_HI,
        preferred_element_type=jnp.float32)
    A = jnp.dot(table_g, W1a_ref[...], preferred_element_type=jnp.float32)
    Brep = jnp.dot(rep, W1b_ref[...],
                   preferred_element_type=jnp.float32) + b1_ref[...]
    AB = jnp.concatenate([A, Brep], axis=0)
    AB_hi = AB.astype(jnp.bfloat16)
    AB_lo = (AB - AB_hi.astype(jnp.float32)).astype(jnp.bfloat16)
    hx = jax.lax.dot_general(cmb, AB_hi, (((0,), (0,)), ((), ())),
                             preferred_element_type=jnp.float32)
    hy = jax.lax.dot_general(cmb, AB_lo, (((0,), (0,)), ((), ())),
                             preferred_element_type=jnp.float32)
    hn = jnp.dot(noise_ref[...].astype(jnp.bfloat16),
                 W1c_ref[...].astype(jnp.bfloat16),
                 preferred_element_type=jnp.float32)
    h = jax.nn.relu(hx + hy + hn)
    out_ref[...] = jnp.dot(h.astype(jnp.bfloat16),
                           W2_ref[...].astype(jnp.bfloat16),
                           preferred_element_type=jnp.float32) + b2_ref[...]


@jax.jit
def _run(objs, noise, obj_to_img, obj_emb, gconv_W, gconv_b, att_W,
         box_W1, box_b1, box_W2, box_b2):
    objs_r = objs.astype(jnp.int32).reshape(1, O_N)
    oti_r = obj_to_img.astype(jnp.int32).reshape(1, O_N)
    obj_emb_p = jnp.pad(obj_emb, ((0, KPAD - NUM_OBJS_P1), (0, 0)))

    def full(shape, idx=None):
        if idx is None:
            idx = tuple(0 for _ in shape)
        return pl.BlockSpec(shape, lambda s, _i=idx: _i)

    out = pl.pallas_call(
        _mono_kernel,
        grid=(1,),
        in_specs=[
            full((1, O_N)), full((1, O_N)), full((O_N, NOISE_DIM)),
            full((KPAD, EMB)), full((EMB, GDIM)), full((1, GDIM)),
            full((GDIM, GDIM)),
            full((GDIM, HID)),
            full((GDIM, HID), (1, 0)),
            full((NOISE_DIM, HID), (4, 0)),
            full((1, HID)),
            full((HID, 4)), full((1, 4)),
        ],
        out_specs=full((O_N, 4)),
        out_shape=jax.ShapeDtypeStruct((O_N, 4), jnp.float32),
    )(objs_r, oti_r, noise, obj_emb_p, gconv_W, gconv_b.reshape(1, GDIM),
      att_W, box_W1, box_W1, box_W1, box_b1.reshape(1, HID), box_W2,
      box_b2.reshape(1, 4))

    return out


def kernel(objs, triples, noise, obj_to_img, obj_emb, pred_emb, gconv_W,
           gconv_b, att_W, box_W1, box_b1, box_W2, box_b2):
    del triples, pred_emb
    return _run(objs, noise, obj_to_img, obj_emb, gconv_W, gconv_b, att_W,
                box_W1, box_b1, box_W2, box_b2)


# speedup vs baseline: 1.2335x; 1.2335x over previous
import jax
import jax.numpy as jnp
from jax.experimental import pallas as pl

O_N = 10000
NUM_OBJS_P1 = 180
NIMG = 8
EMB = 128
GDIM = 128
HID = 512
NOISE_DIM = 64

KPAD = 184
CROWS = KPAD + NIMG + NOISE_DIM

_HI = jax.lax.Precision.HIGHEST


def _mono_kernel(objs_ref, oti_ref, noiseT_ref, obj_emb_ref, gconv_W_ref,
                 gconv_b_ref, att_W_ref, W1a_ref, W1b_ref, W1c_ref, b1_ref,
                 W2_ref, b2_ref, out_ref):
    objs_l = objs_ref[...]
    oti_l = oti_ref[...]
    ohT_obj = (jax.lax.broadcasted_iota(jnp.int32, (KPAD, O_N), 0)
               == objs_l).astype(jnp.bfloat16)
    ohT_img = (jax.lax.broadcasted_iota(jnp.int32, (NIMG, O_N), 0)
               == oti_l).astype(jnp.bfloat16)
    histT = jax.lax.dot_general(ohT_obj, ohT_img, (((1,), (1,)), ((), ())),
                                preferred_element_type=jnp.float32)
    table_g = jnp.dot(obj_emb_ref[...], gconv_W_ref[...],
                      preferred_element_type=jnp.float32) + gconv_b_ref[...]
    table_a = jnp.dot(table_g, att_W_ref[...],
                      preferred_element_type=jnp.float32)
    counts = jax.lax.dot_general(
        histT, jnp.ones((KPAD, 1), jnp.float32),
        (((0,), (0,)), ((), ())), preferred_element_type=jnp.float32)
    counts = jnp.where(counts > 0.0, counts, 1.0)
    gc = jax.lax.dot_general(
        histT, table_a, (((0,), (0,)), ((), ())), precision=_HI,
        preferred_element_type=jnp.float32) / counts
    tg = jnp.tanh(gc)
    sig = jax.nn.sigmoid(jax.lax.dot_general(
        table_g, tg, (((1,), (1,)), ((), ())), precision=_HI,
        preferred_element_type=jnp.float32))
    w = histT * sig
    rep = jax.lax.dot_general(
        w, table_g, (((0,), (0,)), ((), ())), precision=_HI,
        preferred_element_type=jnp.float32)
    A = jnp.dot(table_g, W1a_ref[...], preferred_element_type=jnp.float32)
    Brep = jnp.dot(rep, W1b_ref[...],
                   preferred_element_type=jnp.float32) + b1_ref[...]
    AB = jnp.concatenate([A, Brep], axis=0)
    AB_hi = AB.astype(jnp.bfloat16)
    AB_lo = (AB - AB_hi.astype(jnp.float32)).astype(jnp.bfloat16)
    W1c_hi = W1c_ref[...].astype(jnp.bfloat16)
    CC2 = jnp.concatenate(
        [AB_hi, W1c_hi, AB_lo, jnp.zeros((NOISE_DIM, HID), jnp.bfloat16)],
        axis=0)
    noiseT = noiseT_ref[...]
    M2 = jnp.concatenate(
        [ohT_obj, ohT_img, noiseT, ohT_obj, ohT_img, noiseT], axis=0)
    h = jax.nn.relu(jax.lax.dot_general(
        M2, CC2, (((0,), (0,)), ((), ())),
        preferred_element_type=jnp.float32))
    out_ref[...] = jnp.dot(h.astype(jnp.bfloat16), W2_ref[...],
                           preferred_element_type=jnp.float32) + b2_ref[...]


@jax.jit
def _run(objs, noise, obj_to_img, obj_emb, gconv_W, gconv_b, att_W,
         box_W1, box_b1, box_W2, box_b2):
    objs_r = objs.astype(jnp.int32).reshape(1, O_N)
    oti_r = obj_to_img.astype(jnp.int32).reshape(1, O_N)
    obj_emb_p = jnp.pad(obj_emb, ((0, KPAD - NUM_OBJS_P1), (0, 0)))
    noiseT = noise.astype(jnp.bfloat16).T
    W2_bf = box_W2.astype(jnp.bfloat16)

    def full(shape, idx=None):
        if idx is None:
            idx = tuple(0 for _ in shape)
        return pl.BlockSpec(shape, lambda s, _i=idx: _i)

    out = pl.pallas_call(
        _mono_kernel,
        grid=(1,),
        in_specs=[
            full((1, O_N)), full((1, O_N)), full((NOISE_DIM, O_N)),
            full((KPAD, EMB)), full((EMB, GDIM)), full((1, GDIM)),
            full((GDIM, GDIM)),
            full((GDIM, HID)),
            full((GDIM, HID), (1, 0)),
            full((NOISE_DIM, HID), (4, 0)),
            full((1, HID)),
            full((HID, 4)), full((1, 4)),
        ],
        out_specs=full((O_N, 4)),
        out_shape=jax.ShapeDtypeStruct((O_N, 4), jnp.float32),
    )(objs_r, oti_r, noiseT, obj_emb_p, gconv_W, gconv_b.reshape(1, GDIM),
      att_W, box_W1, box_W1, box_W1, box_b1.reshape(1, HID), W2_bf,
      box_b2.reshape(1, 4))

    return out


def kernel(objs, triples, noise, obj_to_img, obj_emb, pred_emb, gconv_W,
           gconv_b, att_W, box_W1, box_b1, box_W2, box_b2):
    del triples, pred_emb
    return _run(objs, noise, obj_to_img, obj_emb, gconv_W, gconv_b, att_W,
                box_W1, box_b1, box_W2, box_b2)
